# baseline (speedup 1.0000x reference)
import functools

import jax
import jax.numpy as jnp
from jax import lax
from jax.experimental import pallas as pl
from jax.experimental.pallas import tpu as pltpu
from jax.experimental.pallas import tpu_sc as plsc

_NC, _NS, _L = 2, 16, 16
_NW = _NC * _NS
_D = 16
_CB = 8


def _gather_body(idx_hbm, table_hbm, out_hbm, idxv, rowsv, sem,
                 *, hist, rows_per_worker):
    wid = lax.axis_index("s") * _NC + lax.axis_index("c")
    n_chunks = rows_per_worker // _CB

    def chunk_body(c, _):
        b0 = wid * rows_per_worker + c * _CB
        pltpu.sync_copy(idx_hbm.at[pl.ds(b0, _CB)], idxv)
        for j in range(_CB):
            pltpu.async_copy(table_hbm.at[idxv.at[j]],
                             rowsv.at[pl.ds(j * hist, hist)], sem)
        for j in range(_CB):
            pltpu.make_async_copy(table_hbm.at[idxv.at[j]],
                                  rowsv.at[pl.ds(j * hist, hist)], sem).wait()
        pltpu.sync_copy(rowsv, out_hbm.at[pl.ds(b0 * hist, _CB * hist)])
        return 0

    lax.fori_loop(0, n_chunks, chunk_body, 0)


@jax.jit
def kernel(input, embedding_weight):
    B, H = input.shape
    n_emb, two_d = embedding_weight.shape
    d = two_d // 2
    assert d == _D and B % (_NW * _CB) == 0
    rows_per_worker = B // _NW
    table = embedding_weight.reshape(n_emb * 2, d)
    idx2 = input.astype(jnp.int32) * 2

    mesh = plsc.VectorSubcoreMesh(core_axis_name="c", subcore_axis_name="s")
    out = pl.kernel(
        functools.partial(_gather_body, hist=H,
                          rows_per_worker=rows_per_worker),
        out_type=jax.ShapeDtypeStruct((B * H, d), jnp.float32),
        mesh=mesh,
        compiler_params=pltpu.CompilerParams(use_tc_tiling_on_sc=False),
        scratch_types=[
            pltpu.VMEM((_CB, H), jnp.int32),
            pltpu.VMEM((_CB * H, _D), jnp.float32),
            pltpu.SemaphoreType.DMA,
        ],
    )(idx2, table)
    return out.reshape(B, H, d)


# speedup vs baseline: 1.2520x; 1.2520x over previous
import functools

import jax
import jax.numpy as jnp
from jax import lax
from jax.experimental import pallas as pl
from jax.experimental.pallas import tpu as pltpu
from jax.experimental.pallas import tpu_sc as plsc

_NC, _NS, _L = 2, 16, 16
_NW = _NC * _NS
_D = 16
_CB = 8


def _gather_body(idx_hbm, table_hbm, out_hbm, idxv, rowsv, sem,
                 *, rows_per_worker):
    wid = lax.axis_index("s") * _NC + lax.axis_index("c")
    n_chunks = rows_per_worker // _CB

    def chunk_body(c, _):
        b0 = wid * rows_per_worker + c * _CB
        pltpu.sync_copy(idx_hbm.at[pl.ds(b0, _CB)], idxv)
        for j in range(_CB):
            pltpu.async_copy(table_hbm.at[idxv.at[j]], rowsv.at[j], sem)
        for j in range(_CB):
            pltpu.make_async_copy(table_hbm.at[idxv.at[j]], rowsv.at[j],
                                  sem).wait()
        pltpu.sync_copy(rowsv, out_hbm.at[pl.ds(b0, _CB)])
        return 0

    lax.fori_loop(0, n_chunks, chunk_body, 0)


@jax.jit
def kernel(input, embedding_weight):
    B, H = input.shape
    n_emb, two_d = embedding_weight.shape
    d = two_d // 2
    assert d == _D and B % (_NW * _CB) == 0
    rows_per_worker = B // _NW
    mu_table = embedding_weight[:, :d]

    mesh = plsc.VectorSubcoreMesh(core_axis_name="c", subcore_axis_name="s")
    out = pl.kernel(
        functools.partial(_gather_body, rows_per_worker=rows_per_worker),
        out_type=jax.ShapeDtypeStruct((B, H, d), jnp.float32),
        mesh=mesh,
        compiler_params=pltpu.CompilerParams(use_tc_tiling_on_sc=False),
        scratch_types=[
            pltpu.VMEM((_CB, H), jnp.int32),
            pltpu.VMEM((_CB, H, _D), jnp.float32),
            pltpu.SemaphoreType.DMA,
        ],
    )(input.astype(jnp.int32), mu_table)
    return out


# speedup vs baseline: 1.2802x; 1.0226x over previous
import functools

import jax
import jax.numpy as jnp
from jax import lax
from jax.experimental import pallas as pl
from jax.experimental.pallas import tpu as pltpu
from jax.experimental.pallas import tpu_sc as plsc

_NC, _NS, _L = 2, 16, 16
_NW = _NC * _NS
_D = 16
_CB = 16
_FB = 8


def _gather_body(idx_hbm, table_hbm, out_hbm, idxv, rowsv, sem,
                 *, rows_per_worker):
    wid = lax.axis_index("s") * _NC + lax.axis_index("c")
    n_chunks = rows_per_worker // _CB

    def chunk_body(c, _):
        b0 = wid * rows_per_worker + c * _CB
        pltpu.sync_copy(idx_hbm.at[pl.ds(b0, _CB)], idxv)
        for h in range(_CB // _FB):
            for j in range(h * _FB, (h + 1) * _FB):
                pltpu.async_copy(table_hbm.at[idxv.at[j]], rowsv.at[j], sem)
            for j in range(h * _FB, (h + 1) * _FB):
                pltpu.make_async_copy(table_hbm.at[idxv.at[j]], rowsv.at[j],
                                      sem).wait()
        pltpu.sync_copy(rowsv, out_hbm.at[pl.ds(b0, _CB)])
        return 0

    lax.fori_loop(0, n_chunks, chunk_body, 0)


@jax.jit
def kernel(input, embedding_weight):
    B, H = input.shape
    n_emb, two_d = embedding_weight.shape
    d = two_d // 2
    assert d == _D and B % (_NW * _CB) == 0
    rows_per_worker = B // _NW
    mu_table = embedding_weight[:, :d]

    mesh = plsc.VectorSubcoreMesh(core_axis_name="c", subcore_axis_name="s")
    out = pl.kernel(
        functools.partial(_gather_body, rows_per_worker=rows_per_worker),
        out_type=jax.ShapeDtypeStruct((B, H, d), jnp.float32),
        mesh=mesh,
        compiler_params=pltpu.CompilerParams(use_tc_tiling_on_sc=False),
        scratch_types=[
            pltpu.VMEM((_CB, H), jnp.int32),
            pltpu.VMEM((_CB, H, _D), jnp.float32),
            pltpu.SemaphoreType.DMA,
        ],
    )(input.astype(jnp.int32), mu_table)
    return out
